# baseline (speedup 1.0000x reference)
import functools
import math

import jax
import jax.numpy as jnp
import numpy as np
from jax import lax
from jax.experimental import pallas as pl
from jax.experimental.pallas import tpu as pltpu
from jax.experimental.pallas import tpu_sc as plsc

_ALPHA = 0.1
_THETA = 0.5
_NC = 2
_NS = 16
_CHUNK = 128


_NBUF = 2
_SPLIT = 4


@functools.lru_cache(maxsize=None)
def _make_sc_spmm(n_pad, d, cpw, passes):
    nw = _NC * _NS
    rows_per_sub = n_pad // _NS
    cpp = cpw // passes
    groups = cpp // _NBUF
    mesh = plsc.VectorSubcoreMesh(core_axis_name="c", subcore_axis_name="s")

    @functools.partial(
        pl.kernel,
        out_type=jax.ShapeDtypeStruct((_NC, n_pad, d), jnp.float32),
        mesh=mesh,
        scratch_types=[
            pltpu.VMEM((cpp, _CHUNK), jnp.int32),
            pltpu.VMEM((cpp, _CHUNK), jnp.int32),
            pltpu.VMEM((_NBUF, _CHUNK, d), jnp.float32),
            pltpu.VMEM_SHARED((n_pad, d), jnp.float32),
        ] + [pltpu.SemaphoreType.DMA] * (_NBUF * _SPLIT),
    )
    def sc_spmm(xc_hbm, src_hbm, dst_hbm, zeros_hbm, out_hbm,
                sidx, didx, rows, accum, *gsems):
        c = lax.axis_index("c")
        s = lax.axis_index("s")
        wid = s * _NC + c
        pltpu.sync_copy(zeros_hbm, accum.at[pl.ds(s * rows_per_sub, rows_per_sub)])
        plsc.subcore_barrier()

        sub = _CHUNK // _SPLIT

        def g_start(j, b):
            for k in range(_SPLIT):
                pltpu.async_copy(
                    xc_hbm.at[sidx.at[j, pl.ds(k * sub, sub)]],
                    rows.at[b, pl.ds(k * sub, sub)],
                    gsems[b * _SPLIT + k])

        def g_wait(j, b):
            for k in range(_SPLIT):
                pltpu.make_async_copy(
                    xc_hbm.at[sidx.at[j, pl.ds(k * sub, sub)]],
                    rows.at[b, pl.ds(k * sub, sub)],
                    gsems[b * _SPLIT + k]).wait()

        def step(j, b, last):
            if not last:
                g_start(j + 1, 1 - b)
            g_wait(j, b)
            pltpu.sync_copy(rows.at[b], accum.at[didx.at[j]], add=True)

        for p in range(passes):
            pltpu.sync_copy(src_hbm.at[pl.ds(wid * cpw + p * cpp, cpp)], sidx)
            pltpu.sync_copy(dst_hbm.at[pl.ds(wid * cpw + p * cpp, cpp)], didx)

            g_start(0, 0)

            def body(m, carry):
                for b in range(_NBUF):
                    step(m * _NBUF + b, b, last=False)
                return carry

            lax.fori_loop(0, groups - 1, body, 0)

            for b in range(_NBUF):
                j = cpp - _NBUF + b
                step(j, b, last=(j == cpp - 1))

        plsc.subcore_barrier()
        pltpu.sync_copy(
            accum.at[pl.ds(s * rows_per_sub, rows_per_sub)],
            out_hbm.at[c, pl.ds(s * rows_per_sub, rows_per_sub)],
        )

    return sc_spmm


def _pick_block(n):
    for bn in (1024, 1000, 800, 640, 512, 400, 250, 200, 128, 8, 1):
        if n % bn == 0:
            return bn
    return 1


def _linear_body(x_ref, w_ref, b_ref, o_ref, *, relu):
    y = jnp.dot(x_ref[...], w_ref[...], preferred_element_type=jnp.float32)
    y = y + b_ref[...]
    o_ref[...] = jnp.maximum(y, 0.0) if relu else y


def _linear(x, w, b, relu):
    n, d_in = x.shape
    d_out = w.shape[1]
    bn = _pick_block(n)
    return pl.pallas_call(
        functools.partial(_linear_body, relu=relu),
        grid=(n // bn,),
        in_specs=[
            pl.BlockSpec((bn, d_in), lambda i: (i, 0)),
            pl.BlockSpec((d_in, d_out), lambda i: (0, 0)),
            pl.BlockSpec((1, d_out), lambda i: (0, 0)),
        ],
        out_specs=pl.BlockSpec((bn, d_out), lambda i: (i, 0)),
        out_shape=jax.ShapeDtypeStruct((n, d_out), jnp.float32),
    )(x, w, b.reshape(1, d_out))


def _combine_body(p_ref, x0_ref, w_ref, o_ref, *, beta):
    agg = p_ref[0] + p_ref[1]
    s = (1.0 - _ALPHA) * agg + _ALPHA * x0_ref[...]
    y = (1.0 - beta) * s + beta * jnp.dot(
        s, w_ref[...], preferred_element_type=jnp.float32)
    o_ref[...] = jnp.maximum(y, 0.0)


def _combine(partial, x0, w, beta):
    n, d = x0.shape
    bn = _pick_block(n)
    return pl.pallas_call(
        functools.partial(_combine_body, beta=beta),
        grid=(n // bn,),
        in_specs=[
            pl.BlockSpec((_NC, bn, d), lambda i: (0, i, 0)),
            pl.BlockSpec((bn, d), lambda i: (i, 0)),
            pl.BlockSpec((d, d), lambda i: (0, 0)),
        ],
        out_specs=pl.BlockSpec((bn, d), lambda i: (i, 0)),
        out_shape=jax.ShapeDtypeStruct((n, d), jnp.float32),
    )(partial, x0, w)


def kernel(x, edge_index, lin0_W, lin0_b, convW, lin1_W, lin1_b):
    n, d = x.shape
    num_layers = convW.shape[0]
    e = edge_index.shape[1]
    nw = _NC * _NS

    cpw = ((math.ceil(e / (_CHUNK * nw)) + 7) // 8) * 8
    e_pad = cpw * nw * _CHUNK
    n_pad = ((n + 1 + _NS * 8 - 1) // (_NS * 8)) * (_NS * 8)

    src = edge_index[0]
    dst = edge_index[1]
    pad = e_pad - e
    if pad:
        src = jnp.concatenate([src, jnp.zeros((pad,), jnp.int32)])
        dst = jnp.concatenate([dst, jnp.full((pad,), n, jnp.int32)])
    src2d = src.reshape(nw * cpw, _CHUNK)
    dst2d = dst.reshape(nw * cpw, _CHUNK)
    zeros = jnp.zeros((n_pad // _NS, d), jnp.float32)

    spmem_budget = 2_000_000
    per_tile_free = (spmem_budget - n_pad * d) // _NS - _NBUF * _CHUNK * d
    passes = 1
    while cpw % passes or (cpw // passes) % 8 or 2 * (cpw // passes) * _CHUNK > per_tile_free:
        passes += 1
        if passes > cpw:
            raise ValueError("no feasible index staging split")

    sc_spmm = _make_sc_spmm(n_pad, d, cpw, passes)

    h = _linear(x, lin0_W, lin0_b, relu=True)
    x0 = h
    xc = h
    for i in range(num_layers):
        partial = sc_spmm(xc, src2d, dst2d, zeros)
        beta = float(np.log(_THETA / (i + 1) + 1.0))
        xc = _combine(partial, x0, convW[i], beta)
    return _linear(xc, lin1_W, lin1_b, relu=False)


# speedup vs baseline: 1.0120x; 1.0120x over previous
import functools
import math

import jax
import jax.numpy as jnp
import numpy as np
from jax import lax
from jax.experimental import pallas as pl
from jax.experimental.pallas import tpu as pltpu
from jax.experimental.pallas import tpu_sc as plsc

_ALPHA = 0.1
_THETA = 0.5
_NC = 2
_NS = 16
_CHUNK = 128


_NBUF = 2


@functools.lru_cache(maxsize=None)
def _make_sc_spmm(n_pad, d, cpw, passes):
    nw = _NC * _NS
    rows_per_sub = n_pad // _NS
    cpp = cpw // passes
    groups = cpp // _NBUF
    mesh = plsc.VectorSubcoreMesh(core_axis_name="c", subcore_axis_name="s")

    @functools.partial(
        pl.kernel,
        out_type=jax.ShapeDtypeStruct((_NC, n_pad, d), jnp.float32),
        mesh=mesh,
        scratch_types=[
            pltpu.VMEM((cpp, _CHUNK), jnp.int32),
            pltpu.VMEM((cpp, _CHUNK), jnp.int32),
            pltpu.VMEM((_NBUF, _CHUNK, d), jnp.float32),
            pltpu.VMEM_SHARED((n_pad, d), jnp.float32),
        ] + [pltpu.SemaphoreType.DMA] * _NBUF,
    )
    def sc_spmm(xc_hbm, src_hbm, dst_hbm, zeros_hbm, out_hbm,
                sidx, didx, rows, accum, *gsems):
        c = lax.axis_index("c")
        s = lax.axis_index("s")
        wid = s * _NC + c
        pltpu.sync_copy(zeros_hbm, accum.at[pl.ds(s * rows_per_sub, rows_per_sub)])
        plsc.subcore_barrier()

        def g_start(j, b):
            pltpu.async_copy(xc_hbm.at[sidx.at[j]], rows.at[b], gsems[b])

        def g_wait(j, b):
            pltpu.make_async_copy(xc_hbm.at[sidx.at[j]], rows.at[b], gsems[b]).wait()

        def step(j, b, last):
            if not last:
                g_start(j + 1, 1 - b)
            g_wait(j, b)
            pltpu.sync_copy(rows.at[b], accum.at[didx.at[j]], add=True)

        for p in range(passes):
            pltpu.sync_copy(src_hbm.at[pl.ds(wid * cpw + p * cpp, cpp)], sidx)
            pltpu.sync_copy(dst_hbm.at[pl.ds(wid * cpw + p * cpp, cpp)], didx)

            g_start(0, 0)

            def body(m, carry):
                for b in range(_NBUF):
                    step(m * _NBUF + b, b, last=False)
                return carry

            lax.fori_loop(0, groups - 1, body, 0)

            for b in range(_NBUF):
                j = cpp - _NBUF + b
                step(j, b, last=(j == cpp - 1))

        plsc.subcore_barrier()
        pltpu.sync_copy(
            accum.at[pl.ds(s * rows_per_sub, rows_per_sub)],
            out_hbm.at[c, pl.ds(s * rows_per_sub, rows_per_sub)],
        )

    return sc_spmm


def _pick_block(n):
    for bn in (1024, 1000, 800, 640, 512, 400, 250, 200, 128, 8, 1):
        if n % bn == 0:
            return bn
    return 1


def _linear_body(x_ref, w_ref, b_ref, o_ref, *, relu):
    y = jnp.dot(x_ref[...], w_ref[...], preferred_element_type=jnp.float32)
    y = y + b_ref[...]
    o_ref[...] = jnp.maximum(y, 0.0) if relu else y


def _linear(x, w, b, relu):
    n, d_in = x.shape
    d_out = w.shape[1]
    bn = _pick_block(n)
    return pl.pallas_call(
        functools.partial(_linear_body, relu=relu),
        grid=(n // bn,),
        in_specs=[
            pl.BlockSpec((bn, d_in), lambda i: (i, 0)),
            pl.BlockSpec((d_in, d_out), lambda i: (0, 0)),
            pl.BlockSpec((1, d_out), lambda i: (0, 0)),
        ],
        out_specs=pl.BlockSpec((bn, d_out), lambda i: (i, 0)),
        out_shape=jax.ShapeDtypeStruct((n, d_out), jnp.float32),
    )(x, w, b.reshape(1, d_out))


def _combine_body(p_ref, x0_ref, w_ref, o_ref, *, beta):
    agg = p_ref[0] + p_ref[1]
    s = (1.0 - _ALPHA) * agg + _ALPHA * x0_ref[...]
    y = (1.0 - beta) * s + beta * jnp.dot(
        s, w_ref[...], preferred_element_type=jnp.float32)
    o_ref[...] = jnp.maximum(y, 0.0)


def _combine(partial, x0, w, beta):
    n, d = x0.shape
    bn = _pick_block(n)
    return pl.pallas_call(
        functools.partial(_combine_body, beta=beta),
        grid=(n // bn,),
        in_specs=[
            pl.BlockSpec((_NC, bn, d), lambda i: (0, i, 0)),
            pl.BlockSpec((bn, d), lambda i: (i, 0)),
            pl.BlockSpec((d, d), lambda i: (0, 0)),
        ],
        out_specs=pl.BlockSpec((bn, d), lambda i: (i, 0)),
        out_shape=jax.ShapeDtypeStruct((n, d), jnp.float32),
    )(partial, x0, w)


def kernel(x, edge_index, lin0_W, lin0_b, convW, lin1_W, lin1_b):
    n, d = x.shape
    num_layers = convW.shape[0]
    e = edge_index.shape[1]
    nw = _NC * _NS

    cpw = ((math.ceil(e / (_CHUNK * nw)) + 7) // 8) * 8
    e_pad = cpw * nw * _CHUNK
    n_pad = ((n + 1 + _NS * 8 - 1) // (_NS * 8)) * (_NS * 8)

    src = edge_index[0]
    dst = edge_index[1]
    pad = e_pad - e
    if pad:
        src = jnp.concatenate([src, jnp.zeros((pad,), jnp.int32)])
        dst = jnp.concatenate([dst, jnp.full((pad,), n, jnp.int32)])
    src2d = src.reshape(nw * cpw, _CHUNK)
    dst2d = dst.reshape(nw * cpw, _CHUNK)
    zeros = jnp.zeros((n_pad // _NS, d), jnp.float32)

    spmem_budget = 2_000_000
    per_tile_free = (spmem_budget - n_pad * d) // _NS - _NBUF * _CHUNK * d
    passes = 1
    while cpw % passes or (cpw // passes) % 8 or 2 * (cpw // passes) * _CHUNK > per_tile_free:
        passes += 1
        if passes > cpw:
            raise ValueError("no feasible index staging split")

    sc_spmm = _make_sc_spmm(n_pad, d, cpw, passes)

    h = _linear(x, lin0_W, lin0_b, relu=True)
    x0 = h
    xc = h
    for i in range(num_layers):
        partial = sc_spmm(xc, src2d, dst2d, zeros)
        beta = float(np.log(_THETA / (i + 1) + 1.0))
        xc = _combine(partial, x0, convW[i], beta)
    return _linear(xc, lin1_W, lin1_b, relu=False)
